# baseline (speedup 1.0000x reference)
import jax
import jax.numpy as jnp
from jax import lax
from jax.experimental import pallas as pl
from jax.experimental.pallas import tpu as pltpu
from jax.experimental.pallas import tpu_sc as plsc

B = 16384
K = 50
D = 32
NC, NS, L = 2, 16, 16
NW = NC * NS
SPW = B // NW
CS = 8
IDXC = CS * K
NCH = SPW // CS
NV = IDXC // L

_mesh = plsc.VectorSubcoreMesh(core_axis_name="c", subcore_axis_name="s")


def _body(x_hbm, w_hbm, out_hbm, idx_all, ge0, go0, ge1, go1, re0, ro0, re1,
          ro1, out_buf, sem0, sem1):
    wid = lax.axis_index("s") * NC + lax.axis_index("c")
    base = wid * (SPW * K)

    pltpu.sync_copy(x_hbm.at[pl.ds(base, SPW * K)], idx_all)

    def build(c, ge, go):
        off = pl.multiple_of(c * IDXC, 8)
        for v in range(NV):
            iv = idx_all[pl.ds(off + v * L, L)]
            iv2 = iv + iv
            ge[pl.ds(v * L, L)] = iv2
            go[pl.ds(v * L, L)] = iv2 + 1

    def start(ge, go, re, ro, sem):
        pltpu.async_copy(w_hbm.at[ge], re, sem)
        pltpu.async_copy(w_hbm.at[go], ro, sem)

    def wait(ge, go, re, ro, sem):
        pltpu.make_async_copy(w_hbm.at[ge], re, sem).wait()
        pltpu.make_async_copy(w_hbm.at[go], ro, sem).wait()

    def reduce_chunk(re, ro, c):
        for s in range(CS):
            a0 = re[s * K, :]
            a1 = ro[s * K, :]
            for j in range(1, K):
                a0 = a0 + re[s * K + j, :]
                a1 = a1 + ro[s * K + j, :]
            row = c * CS + s
            out_buf[row, 0:L] = a0
            out_buf[row, L:D] = a1

    build(0, ge0, go0)
    start(ge0, go0, re0, ro0, sem0)

    def pair(i, carry):
        c0 = i * 2
        build(c0 + 1, ge1, go1)
        start(ge1, go1, re1, ro1, sem1)
        wait(ge0, go0, re0, ro0, sem0)
        reduce_chunk(re0, ro0, c0)
        build(c0 + 2, ge0, go0)
        start(ge0, go0, re0, ro0, sem0)
        wait(ge1, go1, re1, ro1, sem1)
        reduce_chunk(re1, ro1, c0 + 1)
        return carry

    lax.fori_loop(0, NCH // 2 - 1, pair, 0)
    build(NCH - 1, ge1, go1)
    start(ge1, go1, re1, ro1, sem1)
    wait(ge0, go0, re0, ro0, sem0)
    reduce_chunk(re0, ro0, NCH - 2)
    wait(ge1, go1, re1, ro1, sem1)
    reduce_chunk(re1, ro1, NCH - 1)

    pltpu.sync_copy(out_buf, out_hbm.at[pl.ds(wid * SPW, SPW)])


_emb_sum = pl.kernel(
    _body,
    out_type=jax.ShapeDtypeStruct((B, D), jnp.float32),
    mesh=_mesh,
    scratch_types=[
        pltpu.VMEM((SPW * K,), jnp.int32),
        pltpu.VMEM((IDXC,), jnp.int32),
        pltpu.VMEM((IDXC,), jnp.int32),
        pltpu.VMEM((IDXC,), jnp.int32),
        pltpu.VMEM((IDXC,), jnp.int32),
        pltpu.VMEM((IDXC, L), jnp.float32),
        pltpu.VMEM((IDXC, L), jnp.float32),
        pltpu.VMEM((IDXC, L), jnp.float32),
        pltpu.VMEM((IDXC, L), jnp.float32),
        pltpu.VMEM((SPW, D), jnp.float32),
        pltpu.SemaphoreType.DMA,
        pltpu.SemaphoreType.DMA,
    ],
    compiler_params=pltpu.CompilerParams(use_tc_tiling_on_sc=False),
)


def kernel(x, weight):
    return _emb_sum(x.reshape(-1), weight.reshape(2 * 1000000, L))
